# baseline (speedup 1.0000x reference)
import functools

import jax
import jax.numpy as jnp
from jax import lax
from jax.experimental import pallas as pl
from jax.experimental.pallas import tpu as pltpu
from jax.experimental.pallas import tpu_sc as plsc

S, D = 2048, 768
H, KVH, DH = 12, 4, 64
E, TOPK, F = 8, 2, 1024
EPS = 1e-6
REP = H // KVH
HALF = DH // 2

BQ = 256
NQ = S // BQ
BT = 256
SLOTS = TOPK * S + E * BT
NB = SLOTS // BT

NEG = -1e30


def _kv_body(x_ref, ln1_ref, wk_ref, wkr_ref, wv_ref, cos_ref, sin_ref,
             h_ref, k_ref, v_ref):
    xb = x_ref[...]
    var = jnp.mean(xb * xb, axis=-1, keepdims=True)
    hb = xb * lax.rsqrt(var + EPS) * ln1_ref[...]
    h_ref[...] = hb
    hbb = hb.astype(jnp.bfloat16)
    cos = cos_ref[...]
    sin = sin_ref[...]
    for kv in range(KVH):
        k0 = jnp.dot(hbb, wk_ref[kv], preferred_element_type=jnp.float32)
        kr = jnp.dot(hbb, wkr_ref[kv], preferred_element_type=jnp.float32)
        k_ref[kv] = (k0 * cos + kr * sin).astype(jnp.bfloat16)
        v_ref[kv] = jnp.dot(hbb, wv_ref[kv],
                            preferred_element_type=jnp.float32
                            ).astype(jnp.bfloat16)


def _attn_body(h_ref, x_ref, wq_ref, wqr_ref, cos_ref, sin_ref,
               k_ref, v_ref, wo_ref, o_ref):
    i = pl.program_id(0)
    hbb = h_ref[...].astype(jnp.bfloat16)
    cos = cos_ref[...]
    sin = sin_ref[...]
    row = i * BQ + lax.broadcasted_iota(jnp.int32, (BQ, S), 0)
    col = lax.broadcasted_iota(jnp.int32, (BQ, S), 1)
    causal = col <= row
    acc = x_ref[...]
    for h in range(H):
        kv = h // REP
        q0 = jnp.dot(hbb, wq_ref[h], preferred_element_type=jnp.float32)
        qr = jnp.dot(hbb, wqr_ref[h], preferred_element_type=jnp.float32)
        qh = ((q0 * cos + qr * sin) * (1.0 / 8.0)).astype(jnp.bfloat16)
        scores = lax.dot_general(qh, k_ref[kv],
                                 (((1,), (1,)), ((), ())),
                                 preferred_element_type=jnp.float32)
        scores = jnp.where(causal, scores, NEG)
        m = jnp.max(scores, axis=-1, keepdims=True)
        p = jnp.exp(scores - m)
        p = p / jnp.sum(p, axis=-1, keepdims=True)
        ctx = jnp.dot(p.astype(jnp.bfloat16), v_ref[kv],
                      preferred_element_type=jnp.float32)
        acc = acc + jnp.dot(ctx.astype(jnp.bfloat16), wo_ref[h],
                            preferred_element_type=jnp.float32)
    o_ref[...] = acc


def _router_body(x_ref, ln2_ref, rw_ref, h2_ref, ti_ref, tw_ref):
    xb = x_ref[...]
    var = jnp.mean(xb * xb, axis=-1, keepdims=True)
    hb = xb * lax.rsqrt(var + EPS) * ln2_ref[...]
    h2_ref[...] = hb
    logits = jnp.dot(hb, rw_ref[...], preferred_element_type=jnp.float32)
    m = jnp.max(logits, axis=-1, keepdims=True)
    pe = jnp.exp(logits - m)
    probs = pe / jnp.sum(pe, axis=-1, keepdims=True)
    ei = lax.broadcasted_iota(jnp.int32, (BQ, E), 1)
    m1 = jnp.max(probs, axis=-1, keepdims=True)
    i1 = jnp.min(jnp.where(probs >= m1, ei, E), axis=-1, keepdims=True)
    p2 = jnp.where(ei == i1, -1.0, probs)
    m2 = jnp.max(p2, axis=-1, keepdims=True)
    i2 = jnp.min(jnp.where(p2 >= m2, ei, E), axis=-1, keepdims=True)
    den = m1 + m2
    ti_ref[...] = jnp.concatenate([i1, i2], axis=-1)
    tw_ref[...] = jnp.concatenate([m1 / den, m2 / den], axis=-1)


def _ffn_body(be_ref, vl_ref, xs_ref, wg_ref, wu_ref, wd_ref, ws_ref,
              o_ref):
    b = pl.program_id(0)

    @pl.when(vl_ref[b] == 1)
    def _():
        xb = xs_ref[...].astype(jnp.bfloat16)
        g = jnp.dot(xb, wg_ref[0], preferred_element_type=jnp.float32)
        u = jnp.dot(xb, wu_ref[0], preferred_element_type=jnp.float32)
        a = (g * jax.nn.sigmoid(g)) * u
        o = jnp.dot(a.astype(jnp.bfloat16), wd_ref[0],
                    preferred_element_type=jnp.float32)
        o_ref[...] = o * ws_ref[...]


def _rot_w(w, nh):
    w4 = w.reshape(D, nh, DH)
    return jnp.concatenate([-w4[:, :, HALF:], w4[:, :, :HALF]],
                           axis=-1).reshape(D, nh * DH)


def kernel(x, ln1_scale, ln2_scale, wq, wk, wv, wo, router_w,
           w_gate, w_up, w_down):
    f32, bf16 = jnp.float32, jnp.bfloat16
    x2d = x[0]

    inv = 1.0 / (10000.0 ** (jnp.arange(0, HALF, dtype=f32) / HALF))
    ang = jnp.arange(S, dtype=f32)[:, None] * inv[None, :]
    cos64 = jnp.concatenate([jnp.cos(ang), jnp.cos(ang)], axis=-1)
    sin64 = jnp.concatenate([jnp.sin(ang), jnp.sin(ang)], axis=-1)

    wq_t = wq.reshape(D, H, DH).transpose(1, 0, 2).astype(bf16)
    wqr_t = _rot_w(wq, H).reshape(D, H, DH).transpose(1, 0, 2).astype(bf16)
    wk_t = wk.reshape(D, KVH, DH).transpose(1, 0, 2).astype(bf16)
    wkr_t = _rot_w(wk, KVH).reshape(D, KVH, DH).transpose(1, 0, 2).astype(bf16)
    wv_t = wv.reshape(D, KVH, DH).transpose(1, 0, 2).astype(bf16)
    wo_t = wo.reshape(H, DH, D).astype(bf16)
    wg_b = w_gate.astype(bf16)
    wu_b = w_up.astype(bf16)
    wd_b = w_down.astype(bf16)
    ln1 = ln1_scale.reshape(1, D)
    ln2 = ln2_scale.reshape(1, D)

    h, k, v = pl.pallas_call(
        _kv_body,
        grid=(NQ,),
        in_specs=[
            pl.BlockSpec((BQ, D), lambda i: (i, 0)),
            pl.BlockSpec((1, D), lambda i: (0, 0)),
            pl.BlockSpec((KVH, D, DH), lambda i: (0, 0, 0)),
            pl.BlockSpec((KVH, D, DH), lambda i: (0, 0, 0)),
            pl.BlockSpec((KVH, D, DH), lambda i: (0, 0, 0)),
            pl.BlockSpec((BQ, DH), lambda i: (i, 0)),
            pl.BlockSpec((BQ, DH), lambda i: (i, 0)),
        ],
        out_specs=[
            pl.BlockSpec((BQ, D), lambda i: (i, 0)),
            pl.BlockSpec((KVH, BQ, DH), lambda i: (0, i, 0)),
            pl.BlockSpec((KVH, BQ, DH), lambda i: (0, i, 0)),
        ],
        out_shape=[
            jax.ShapeDtypeStruct((S, D), f32),
            jax.ShapeDtypeStruct((KVH, S, DH), bf16),
            jax.ShapeDtypeStruct((KVH, S, DH), bf16),
        ],
    )(x2d, ln1, wk_t, wkr_t, wv_t, cos64, sin64)

    x2 = pl.pallas_call(
        _attn_body,
        grid=(NQ,),
        in_specs=[
            pl.BlockSpec((BQ, D), lambda i: (i, 0)),
            pl.BlockSpec((BQ, D), lambda i: (i, 0)),
            pl.BlockSpec((H, D, DH), lambda i: (0, 0, 0)),
            pl.BlockSpec((H, D, DH), lambda i: (0, 0, 0)),
            pl.BlockSpec((BQ, DH), lambda i: (i, 0)),
            pl.BlockSpec((BQ, DH), lambda i: (i, 0)),
            pl.BlockSpec((KVH, S, DH), lambda i: (0, 0, 0)),
            pl.BlockSpec((KVH, S, DH), lambda i: (0, 0, 0)),
            pl.BlockSpec((H, DH, D), lambda i: (0, 0, 0)),
        ],
        out_specs=pl.BlockSpec((BQ, D), lambda i: (i, 0)),
        out_shape=jax.ShapeDtypeStruct((S, D), f32),
    )(h, x2d, wq_t, wqr_t, cos64, sin64, k, v, wo_t)

    h2, topi, topw = pl.pallas_call(
        _router_body,
        grid=(NQ,),
        in_specs=[
            pl.BlockSpec((BQ, D), lambda i: (i, 0)),
            pl.BlockSpec((1, D), lambda i: (0, 0)),
            pl.BlockSpec((D, E), lambda i: (0, 0)),
        ],
        out_specs=[
            pl.BlockSpec((BQ, D), lambda i: (i, 0)),
            pl.BlockSpec((BQ, TOPK), lambda i: (i, 0)),
            pl.BlockSpec((BQ, TOPK), lambda i: (i, 0)),
        ],
        out_shape=[
            jax.ShapeDtypeStruct((S, D), f32),
            jax.ShapeDtypeStruct((S, TOPK), jnp.int32),
            jax.ShapeDtypeStruct((S, TOPK), f32),
        ],
    )(x2, ln2, router_w)

    e_flat = topi.reshape(-1)
    onehot = (e_flat[:, None] == jnp.arange(E)[None, :]).astype(jnp.int32)
    counts = jnp.sum(onehot, axis=0)
    pc = ((counts + BT - 1) // BT) * BT
    ends = jnp.cumsum(pc)
    starts = ends - pc
    rank = jnp.take_along_axis(jnp.cumsum(onehot, axis=0),
                               e_flat[:, None], axis=1)[:, 0] - 1
    pos = starts[e_flat] + rank
    gather_pair = jnp.zeros((SLOTS,), jnp.int32).at[pos].set(
        jnp.arange(TOPK * S, dtype=jnp.int32))
    bidx = jnp.arange(NB, dtype=jnp.int32) * BT
    blk_e = jnp.searchsorted(ends, bidx, side='right').astype(jnp.int32)
    valid = (bidx < ends[-1]).astype(jnp.int32)
    last_e = jnp.searchsorted(ends, ends[-1] - 1,
                              side='right').astype(jnp.int32)
    blk_e = jnp.where(valid == 1, jnp.minimum(blk_e, E - 1), last_e)

    tok = gather_pair >> 1
    xs = h2[tok]
    w_slot = topw.reshape(-1)[gather_pair].reshape(SLOTS, 1)

    grid_spec = pltpu.PrefetchScalarGridSpec(
        num_scalar_prefetch=2,
        grid=(NB,),
        in_specs=[
            pl.BlockSpec((BT, D), lambda b, be, vl: (b, 0)),
            pl.BlockSpec((1, D, F), lambda b, be, vl: (be[b], 0, 0)),
            pl.BlockSpec((1, D, F), lambda b, be, vl: (be[b], 0, 0)),
            pl.BlockSpec((1, F, D), lambda b, be, vl: (be[b], 0, 0)),
            pl.BlockSpec((BT, 1), lambda b, be, vl: (b, 0)),
        ],
        out_specs=pl.BlockSpec((BT, D), lambda b, be, vl: (b, 0)),
    )
    outw = pl.pallas_call(
        _ffn_body,
        grid_spec=grid_spec,
        out_shape=jax.ShapeDtypeStruct((SLOTS, D), f32),
    )(blk_e, valid, xs, wg_b, wu_b, wd_b, w_slot)

    pos2 = pos.reshape(S, TOPK)
    xout = x2 + outw[pos2[:, 0]] + outw[pos2[:, 1]]

    return xout.reshape(1, S, D)
